# Initial kernel scaffold; baseline (speedup 1.0000x reference)
import jax
import jax.numpy as jnp
from jax.experimental import pallas as pl


def kernel(t, u, edge_index, Wa, ba, Wfc, bfc, Wfh, bfh, Wgc, bgc, Wz, bz):
    raise NotImplementedError("write your pallas kernel here")


# speedup vs baseline: 3.5070x; 3.5070x over previous
import jax
import jax.numpy as jnp
from jax import lax
from jax.experimental import pallas as pl
from jax.experimental.pallas import tpu as pltpu
from jax.experimental.pallas import tpu_sc as plsc

N = 10000
P = 128
Q = 128
E = 320000

NC = 2
NS = 16
HALF = N // NC
NR = 5120
DUMMY = 5100
EPT = E // NS
K = 80
BLK = 2000
CPB = BLK // K
ZR = NR // NS
ZCH = 40
WPT = 312
DEGW = 128


def _sc_body(h_hbm, src_hbm, dst_hbm, z128_hbm, o128_hbm,
             sum_hbm, deg_hbm,
             src_blk, dst_blk, cdst, rows, ones_v, sem, acc, dacc):
    cid = lax.axis_index("c")
    sid = lax.axis_index("s")
    lo = cid * HALF

    pltpu.sync_copy(z128_hbm, acc.at[pl.ds(sid * ZR, ZR)])
    pltpu.sync_copy(z128_hbm, dacc.at[pl.ds(sid * ZR, ZR)])
    pltpu.sync_copy(o128_hbm, ones_v)
    plsc.subcore_barrier()

    def _block(b, _):
        base = sid * EPT + b * BLK
        pltpu.sync_copy(src_hbm.at[pl.ds(base, BLK)], src_blk)
        pltpu.sync_copy(dst_hbm.at[pl.ds(base, BLK)], dst_blk)

        def _trow(r, _):
            for cc in range(K // 16):
                v = dst_blk[pl.ds(r * K + cc * 16, 16)]
                t = v - lo
                m = jnp.logical_and(t >= 0, t < HALF)
                cdst[r, pl.ds(cc * 16, 16)] = jnp.where(m, t, DUMMY)
            return 0
        lax.fori_loop(0, CPB, _trow, 0)

        def _chunk(j, _):
            pltpu.async_copy(h_hbm.at[src_blk.at[pl.ds(j * K, K)]], rows,
                             sem).wait()
            pltpu.sync_copy(rows, acc.at[cdst.at[j]], add=True)
            pltpu.sync_copy(ones_v, dacc.at[cdst.at[j]], add=True)
            return 0
        lax.fori_loop(0, CPB, _chunk, 0)
        return 0
    lax.fori_loop(0, EPT // BLK, _block, 0)
    plsc.subcore_barrier()

    wlo = sid * WPT
    pltpu.sync_copy(acc.at[pl.ds(wlo, WPT)],
                    sum_hbm.at[pl.ds(cid * HALF + wlo, WPT)])
    pltpu.sync_copy(dacc.at[pl.ds(wlo, WPT)],
                    deg_hbm.at[pl.ds(cid * HALF + wlo, WPT)])

    @pl.when(sid == NS - 1)
    def _tail():
        t0 = NS * WPT
        pltpu.sync_copy(acc.at[pl.ds(t0, HALF - t0)],
                        sum_hbm.at[pl.ds(cid * HALF + t0, HALF - t0)])
        pltpu.sync_copy(dacc.at[pl.ds(t0, HALF - t0)],
                        deg_hbm.at[pl.ds(cid * HALF + t0, HALF - t0)])


_sc_call = pl.kernel(
    _sc_body,
    out_type=[
        jax.ShapeDtypeStruct((N, Q), jnp.float32),
        jax.ShapeDtypeStruct((N, DEGW), jnp.float32),
    ],
    mesh=plsc.VectorSubcoreMesh(
        core_axis_name="c", subcore_axis_name="s", num_cores=NC,
        num_subcores=NS),
    scratch_types=[
        pltpu.VMEM((BLK,), jnp.int32),
        pltpu.VMEM((BLK,), jnp.int32),
        pltpu.VMEM((CPB, K), jnp.int32),
        pltpu.VMEM((K, Q), jnp.float32),
        pltpu.VMEM((K, DEGW), jnp.float32),
        pltpu.SemaphoreType.DMA,
        pltpu.VMEM_SHARED((NR, Q), jnp.float32),
        pltpu.VMEM_SHARED((NR, DEGW), jnp.float32),
    ],
)


BR = 1000


def _tc_body(u_ref, n_ref, d_ref,
             wa1, wa2, wfc1, wfc2, wfh1, wfh2, wgc1, wgc2, wz1, wz2,
             ba_r, bfc_r, bfh_r, bgc_r, bz_r, out_ref):
    c = u_ref[:, :P]
    h = u_ref[:, P:]
    nsum = n_ref[...]
    deg = d_ref[:, :1]
    mean = jnp.where(deg > 0, nsum / jnp.maximum(deg, 1.0), 0.0)

    def dot(a, b):
        return lax.dot_general(a, b, (((1,), (0,)), ((), ())),
                               precision=lax.Precision.HIGHEST,
                               preferred_element_type=jnp.float32)

    def sp(x):
        return jnp.maximum(x, 0.0) + jnp.log1p(jnp.exp(-jnp.abs(x)))

    h_ = sp(dot(h, wa1[...]) + dot(mean, wa2[...]) + ba_r[...])
    fc = sp(dot(c, wfc1[...]) + dot(h_, wfc2[...]) + bfc_r[...])
    gc = sp(dot(c, wgc1[...]) + dot(h_, wgc2[...]) + bgc_r[...])
    zz = jnp.tanh(dot(c, wz1[...]) + dot(h_, wz2[...]) + bz_r[...])
    fh = sp(dot(c, wfh1[...]) + dot(h_, wfh2[...]) + bfh_r[...])
    dc = gc * zz - fc * c
    dh = -fh * h
    dc = dc - (jnp.sum(dc * c, axis=1, keepdims=True)
               / jnp.sum(c * c, axis=1, keepdims=True)) * c
    out_ref[:, :P] = dc
    out_ref[:, P:] = dh


def _full(shape):
    return pl.BlockSpec(shape, lambda i: (0,) * len(shape))


_tc_call = pl.pallas_call(
    _tc_body,
    grid=(N // BR,),
    in_specs=[
        pl.BlockSpec((BR, P + Q), lambda i: (i, 0)),
        pl.BlockSpec((BR, Q), lambda i: (i, 0)),
        pl.BlockSpec((BR, DEGW), lambda i: (i, 0)),
    ] + [_full((P, P))] * 10 + [_full((1, P))] * 5,
    out_specs=pl.BlockSpec((BR, P + Q), lambda i: (i, 0)),
    out_shape=jax.ShapeDtypeStruct((N, P + Q), jnp.float32),
)


@jax.jit
def kernel(t, u, edge_index, Wa, ba, Wfc, bfc, Wfh, bfh, Wgc, bgc, Wz, bz):
    del t
    h_tab = u.reshape(2 * N, Q)
    src_flat = edge_index[0] * 2 + 1
    dst_flat = edge_index[1]
    z128 = jnp.zeros((ZR, Q), jnp.float32)
    o128 = jnp.ones((K, DEGW), jnp.float32)
    nsum, deg = _sc_call(h_tab, src_flat, dst_flat, z128, o128)
    out = _tc_call(
        u, nsum, deg,
        Wa[:Q], Wa[Q:], Wfc[:P], Wfc[P:], Wfh[:P], Wfh[P:],
        Wgc[:P], Wgc[P:], Wz[:P], Wz[P:],
        ba.reshape(1, Q), bfc.reshape(1, P), bfh.reshape(1, Q),
        bgc.reshape(1, P), bz.reshape(1, P),
    )
    return out


# speedup vs baseline: 4.0676x; 1.1599x over previous
import jax
import jax.numpy as jnp
from jax import lax
from jax.experimental import pallas as pl
from jax.experimental.pallas import tpu as pltpu
from jax.experimental.pallas import tpu_sc as plsc

N = 10000
P = 128
Q = 128
E = 320000

NC = 2
NS = 16
HALF = N // NC
NR = 5120
DUMMY = 5100
EPT = E // NS
K = 80
BLK = 2000
CPB = BLK // K
ZR = NR // NS
ZCH = 40
WPT = 312
DEGW = 128


def _sc_body(h_hbm, src_hbm, dst_hbm, z128_hbm, o128_hbm,
             sum_hbm, deg_hbm,
             src_blk, dst_blk, cdst, rows0, rows1, ones_v, gsem0, gsem1,
             acc, dacc):
    cid = lax.axis_index("c")
    sid = lax.axis_index("s")
    lo = cid * HALF

    pltpu.sync_copy(z128_hbm, acc.at[pl.ds(sid * ZR, ZR)])
    pltpu.sync_copy(z128_hbm, dacc.at[pl.ds(sid * ZR, ZR)])
    pltpu.sync_copy(o128_hbm, ones_v)
    plsc.subcore_barrier()

    def _block(b, _):
        base = sid * EPT + b * BLK
        pltpu.sync_copy(src_hbm.at[pl.ds(base, BLK)], src_blk)
        pltpu.sync_copy(dst_hbm.at[pl.ds(base, BLK)], dst_blk)

        def _trow(r, _):
            for cc in range(K // 16):
                v = dst_blk[pl.ds(r * K + cc * 16, 16)]
                t = v - lo
                m = jnp.logical_and(t >= 0, t < HALF)
                cdst[r, pl.ds(cc * 16, 16)] = jnp.where(m, t, DUMMY)
            return 0
        lax.fori_loop(0, CPB, _trow, 0)

        pltpu.async_copy(h_hbm.at[src_blk.at[pl.ds(0, K)]], rows0, gsem0)
        pltpu.async_copy(h_hbm.at[src_blk.at[pl.ds(K, K)]], rows1, gsem1)

        def _step(c, buf, sem):
            pltpu.make_async_copy(
                h_hbm.at[src_blk.at[pl.ds(c * K, K)]], buf, sem).wait()
            pltpu.sync_copy(buf, acc.at[cdst.at[c]], add=True)
            pltpu.sync_copy(ones_v, dacc.at[cdst.at[c]], add=True)

            @pl.when(c + 2 < CPB)
            def _next():
                pltpu.async_copy(
                    h_hbm.at[src_blk.at[pl.ds((c + 2) * K, K)]], buf, sem)

        def _pair(p, _):
            _step(2 * p, rows0, gsem0)
            _step(2 * p + 1, rows1, gsem1)
            return 0
        lax.fori_loop(0, CPB // 2, _pair, 0)
        _step(CPB - 1, rows0, gsem0)
        return 0
    lax.fori_loop(0, EPT // BLK, _block, 0)
    plsc.subcore_barrier()

    wlo = sid * WPT
    pltpu.sync_copy(acc.at[pl.ds(wlo, WPT)],
                    sum_hbm.at[pl.ds(cid * HALF + wlo, WPT)])
    pltpu.sync_copy(dacc.at[pl.ds(wlo, WPT)],
                    deg_hbm.at[pl.ds(cid * HALF + wlo, WPT)])

    @pl.when(sid == NS - 1)
    def _tail():
        t0 = NS * WPT
        pltpu.sync_copy(acc.at[pl.ds(t0, HALF - t0)],
                        sum_hbm.at[pl.ds(cid * HALF + t0, HALF - t0)])
        pltpu.sync_copy(dacc.at[pl.ds(t0, HALF - t0)],
                        deg_hbm.at[pl.ds(cid * HALF + t0, HALF - t0)])


_sc_call = pl.kernel(
    _sc_body,
    out_type=[
        jax.ShapeDtypeStruct((N, Q), jnp.float32),
        jax.ShapeDtypeStruct((N, DEGW), jnp.float32),
    ],
    mesh=plsc.VectorSubcoreMesh(
        core_axis_name="c", subcore_axis_name="s", num_cores=NC,
        num_subcores=NS),
    scratch_types=[
        pltpu.VMEM((BLK,), jnp.int32),
        pltpu.VMEM((BLK,), jnp.int32),
        pltpu.VMEM((CPB, K), jnp.int32),
        pltpu.VMEM((K, Q), jnp.float32),
        pltpu.VMEM((K, Q), jnp.float32),
        pltpu.VMEM((K, DEGW), jnp.float32),
        pltpu.SemaphoreType.DMA,
        pltpu.SemaphoreType.DMA,
        pltpu.VMEM_SHARED((NR, Q), jnp.float32),
        pltpu.VMEM_SHARED((NR, DEGW), jnp.float32),
    ],
)


BR = 1000


def _tc_body(u_ref, n_ref, d_ref,
             wa1, wa2, wfc1, wfc2, wfh1, wfh2, wgc1, wgc2, wz1, wz2,
             ba_r, bfc_r, bfh_r, bgc_r, bz_r, out_ref):
    c = u_ref[:, :P]
    h = u_ref[:, P:]
    nsum = n_ref[...]
    deg = d_ref[:, :1]
    mean = jnp.where(deg > 0, nsum / jnp.maximum(deg, 1.0), 0.0)

    def dot(a, b):
        return lax.dot_general(a, b, (((1,), (0,)), ((), ())),
                               precision=lax.Precision.HIGHEST,
                               preferred_element_type=jnp.float32)

    def sp(x):
        return jnp.maximum(x, 0.0) + jnp.log1p(jnp.exp(-jnp.abs(x)))

    h_ = sp(dot(h, wa1[...]) + dot(mean, wa2[...]) + ba_r[...])
    fc = sp(dot(c, wfc1[...]) + dot(h_, wfc2[...]) + bfc_r[...])
    gc = sp(dot(c, wgc1[...]) + dot(h_, wgc2[...]) + bgc_r[...])
    zz = jnp.tanh(dot(c, wz1[...]) + dot(h_, wz2[...]) + bz_r[...])
    fh = sp(dot(c, wfh1[...]) + dot(h_, wfh2[...]) + bfh_r[...])
    dc = gc * zz - fc * c
    dh = -fh * h
    dc = dc - (jnp.sum(dc * c, axis=1, keepdims=True)
               / jnp.sum(c * c, axis=1, keepdims=True)) * c
    out_ref[:, :P] = dc
    out_ref[:, P:] = dh


def _full(shape):
    return pl.BlockSpec(shape, lambda i: (0,) * len(shape))


_tc_call = pl.pallas_call(
    _tc_body,
    grid=(N // BR,),
    in_specs=[
        pl.BlockSpec((BR, P + Q), lambda i: (i, 0)),
        pl.BlockSpec((BR, Q), lambda i: (i, 0)),
        pl.BlockSpec((BR, DEGW), lambda i: (i, 0)),
    ] + [_full((P, P))] * 10 + [_full((1, P))] * 5,
    out_specs=pl.BlockSpec((BR, P + Q), lambda i: (i, 0)),
    out_shape=jax.ShapeDtypeStruct((N, P + Q), jnp.float32),
)


@jax.jit
def kernel(t, u, edge_index, Wa, ba, Wfc, bfc, Wfh, bfh, Wgc, bgc, Wz, bz):
    del t
    h_tab = u.reshape(2 * N, Q)
    src_flat = edge_index[0] * 2 + 1
    dst_flat = edge_index[1]
    z128 = jnp.zeros((ZR, Q), jnp.float32)
    o128 = jnp.ones((K, DEGW), jnp.float32)
    nsum, deg = _sc_call(h_tab, src_flat, dst_flat, z128, o128)
    out = _tc_call(
        u, nsum, deg,
        Wa[:Q], Wa[Q:], Wfc[:P], Wfc[P:], Wfh[:P], Wfh[P:],
        Wgc[:P], Wgc[P:], Wz[:P], Wz[P:],
        ba.reshape(1, Q), bfc.reshape(1, P), bfh.reshape(1, Q),
        bgc.reshape(1, P), bz.reshape(1, P),
    )
    return out


# speedup vs baseline: 8.4310x; 2.0727x over previous
import jax
import jax.numpy as jnp
from jax import lax
from jax.experimental import pallas as pl
from jax.experimental.pallas import tpu as pltpu
from jax.experimental.pallas import tpu_sc as plsc

N = 10000
P = 128
Q = 128
E = 320000

NC = 2
NS = 16
ESC = E // NC
EPT = ESC // NS
K = 80
BLK = 2000
CPB = BLK // K
NBLK = EPT // BLK
NR = 10240
ZR = NR // NS
WPT = 624
DEGW = 128


def _sc_body(h_hbm, src_hbm, dst_hbm, z_hbm, zd_hbm, o_hbm,
             sum_hbm, deg_hbm,
             src_blk, dst_blk, cdst, ones_v, rows0, rows1, gsem0, gsem1,
             acc, dacc):
    cid = lax.axis_index("c")
    sid = lax.axis_index("s")

    pltpu.sync_copy(z_hbm, acc.at[pl.ds(sid * ZR, ZR)])
    pltpu.sync_copy(o_hbm, ones_v)

    @pl.when(sid == 0)
    def _zdeg():
        pltpu.sync_copy(zd_hbm, dacc)

    plsc.subcore_barrier()

    def _block(b, _):
        base = cid * ESC + sid * EPT + b * BLK
        pltpu.sync_copy(src_hbm.at[pl.ds(base, BLK)], src_blk)
        pltpu.sync_copy(dst_hbm.at[pl.ds(base, BLK)], dst_blk)

        def _trow(r, _):
            for cc in range(K // 16):
                v = dst_blk[pl.ds(r * K + cc * 16, 16)]
                cdst[r, pl.ds(cc * 16, 16)] = v
            return 0
        lax.fori_loop(0, CPB, _trow, 0)

        pltpu.async_copy(h_hbm.at[src_blk.at[pl.ds(0, K)]], rows0, gsem0)
        pltpu.async_copy(h_hbm.at[src_blk.at[pl.ds(K, K)]], rows1, gsem1)

        def _step(c, buf, sem):
            pltpu.make_async_copy(
                h_hbm.at[src_blk.at[pl.ds(c * K, K)]], buf, sem).wait()
            pltpu.sync_copy(buf, acc.at[cdst.at[c]], add=True)
            pltpu.sync_copy(ones_v, dacc.at[cdst.at[c]], add=True)

            @pl.when(c + 2 < CPB)
            def _next():
                pltpu.async_copy(
                    h_hbm.at[src_blk.at[pl.ds((c + 2) * K, K)]], buf, sem)

        def _pair(p, _):
            _step(2 * p, rows0, gsem0)
            _step(2 * p + 1, rows1, gsem1)
            return 0
        lax.fori_loop(0, CPB // 2, _pair, 0)
        _step(CPB - 1, rows0, gsem0)
        return 0
    lax.fori_loop(0, NBLK, _block, 0)
    plsc.subcore_barrier()

    wlo = sid * WPT
    pltpu.sync_copy(acc.at[pl.ds(wlo, WPT)],
                    sum_hbm.at[pl.ds(cid * N + wlo, WPT)])

    @pl.when(sid == NS - 1)
    def _tail():
        t0 = NS * WPT
        pltpu.sync_copy(acc.at[pl.ds(t0, N - t0)],
                        sum_hbm.at[pl.ds(cid * N + t0, N - t0)])

    @pl.when(sid == 0)
    def _deg_out():
        pltpu.sync_copy(dacc, deg_hbm.at[pl.ds(cid * NR, NR)])


_sc_call = pl.kernel(
    _sc_body,
    out_type=[
        jax.ShapeDtypeStruct((NC * N, Q), jnp.float32),
        jax.ShapeDtypeStruct((NC * NR,), jnp.float32),
    ],
    mesh=plsc.VectorSubcoreMesh(
        core_axis_name="c", subcore_axis_name="s", num_cores=NC,
        num_subcores=NS),
    scratch_types=[
        pltpu.VMEM((BLK,), jnp.int32),
        pltpu.VMEM((BLK,), jnp.int32),
        pltpu.VMEM((CPB, K), jnp.int32),
        pltpu.VMEM((K,), jnp.float32),
        pltpu.VMEM((K, Q), jnp.float32),
        pltpu.VMEM((K, Q), jnp.float32),
        pltpu.SemaphoreType.DMA,
        pltpu.SemaphoreType.DMA,
        pltpu.VMEM_SHARED((NR, Q), jnp.float32),
        pltpu.VMEM_SHARED((NR,), jnp.float32),
    ],
)


BR = 1000


def _tc_body(u_ref, n0_ref, n1_ref, d_ref,
             wa1, wa2, wfc1, wfc2, wfh1, wfh2, wgc1, wgc2, wz1, wz2,
             ba_r, bfc_r, bfh_r, bgc_r, bz_r, out_ref):
    c = u_ref[:, :P]
    h = u_ref[:, P:]
    nsum = n0_ref[...] + n1_ref[...]
    deg = d_ref[...]
    mean = jnp.where(deg > 0, nsum / jnp.maximum(deg, 1.0), 0.0)

    def dot(a, b):
        return lax.dot_general(a, b, (((1,), (0,)), ((), ())),
                               precision=lax.Precision.HIGHEST,
                               preferred_element_type=jnp.float32)

    def sp(x):
        return jnp.maximum(x, 0.0) + jnp.log1p(jnp.exp(-jnp.abs(x)))

    h_ = sp(dot(h, wa1[...]) + dot(mean, wa2[...]) + ba_r[...])
    fc = sp(dot(c, wfc1[...]) + dot(h_, wfc2[...]) + bfc_r[...])
    gc = sp(dot(c, wgc1[...]) + dot(h_, wgc2[...]) + bgc_r[...])
    zz = jnp.tanh(dot(c, wz1[...]) + dot(h_, wz2[...]) + bz_r[...])
    fh = sp(dot(c, wfh1[...]) + dot(h_, wfh2[...]) + bfh_r[...])
    dc = gc * zz - fc * c
    dh = -fh * h
    dc = dc - (jnp.sum(dc * c, axis=1, keepdims=True)
               / jnp.sum(c * c, axis=1, keepdims=True)) * c
    out_ref[:, :P] = dc
    out_ref[:, P:] = dh


def _full(shape):
    return pl.BlockSpec(shape, lambda i: (0,) * len(shape))


_tc_call = pl.pallas_call(
    _tc_body,
    grid=(N // BR,),
    in_specs=[
        pl.BlockSpec((BR, P + Q), lambda i: (i, 0)),
        pl.BlockSpec((BR, Q), lambda i: (i, 0)),
        pl.BlockSpec((BR, Q), lambda i: (i + N // BR, 0)),
        pl.BlockSpec((BR, 1), lambda i: (i, 0)),
    ] + [_full((P, P))] * 10 + [_full((1, P))] * 5,
    out_specs=pl.BlockSpec((BR, P + Q), lambda i: (i, 0)),
    out_shape=jax.ShapeDtypeStruct((N, P + Q), jnp.float32),
)


@jax.jit
def kernel(t, u, edge_index, Wa, ba, Wfc, bfc, Wfh, bfh, Wgc, bgc, Wz, bz):
    del t
    h_tab = u.reshape(2 * N, Q)
    src_flat = edge_index[0] * 2 + 1
    dst_flat = edge_index[1]
    z = jnp.zeros((ZR, Q), jnp.float32)
    zd = jnp.zeros((NR,), jnp.float32)
    o = jnp.ones((K,), jnp.float32)
    nsum2, deg2 = _sc_call(h_tab, src_flat, dst_flat, z, zd, o)
    deg_col = (deg2[:NR] + deg2[NR:])[:N].reshape(N, 1)
    out = _tc_call(
        u, nsum2, nsum2, deg_col,
        Wa[:Q], Wa[Q:], Wfc[:P], Wfc[P:], Wfh[:P], Wfh[P:],
        Wgc[:P], Wgc[P:], Wz[:P], Wz[P:],
        ba.reshape(1, Q), bfc.reshape(1, P), bfh.reshape(1, Q),
        bgc.reshape(1, P), bz.reshape(1, P),
    )
    return out


# speedup vs baseline: 11.1190x; 1.3188x over previous
import jax
import jax.numpy as jnp
from jax import lax
from jax.experimental import pallas as pl
from jax.experimental.pallas import tpu as pltpu
from jax.experimental.pallas import tpu_sc as plsc

N = 10000
P = 128
Q = 128
E = 320000

NC = 2
NS = 16
ESC = E // NC
EPT = ESC // NS
K = 80
BLK = 2000
CPB = BLK // K
NBLK = EPT // BLK
NR = 10240
ZR = NR // NS
WPT = 624
DEGW = 128


def _sc_body(h_hbm, src_hbm, dst_hbm, z_hbm, zd_hbm, o_hbm,
             sum_hbm, deg_hbm,
             src_blk, dst_blk, cdst, ones_v, rows0, rows1, gsem0, gsem1,
             acc, dacc):
    cid = lax.axis_index("c")
    sid = lax.axis_index("s")

    pltpu.sync_copy(z_hbm, acc.at[pl.ds(sid * ZR, ZR)])
    pltpu.sync_copy(o_hbm, ones_v)

    @pl.when(sid == 0)
    def _zdeg():
        pltpu.sync_copy(zd_hbm, dacc)

    plsc.subcore_barrier()

    def _block(b, _):
        base = cid * ESC + sid * EPT + b * BLK
        pltpu.sync_copy(src_hbm.at[pl.ds(base, BLK)], src_blk)
        pltpu.sync_copy(dst_hbm.at[pl.ds(base, BLK)], dst_blk)

        def _trow(r, _):
            for cc in range(K // 16):
                v = dst_blk[pl.ds(r * K + cc * 16, 16)]
                cdst[r, pl.ds(cc * 16, 16)] = v
            return 0
        lax.fori_loop(0, CPB, _trow, 0)

        pltpu.async_copy(h_hbm.at[src_blk.at[pl.ds(0, K)]], rows0, gsem0)
        pltpu.async_copy(h_hbm.at[src_blk.at[pl.ds(K, K)]], rows1, gsem1)

        def _step(c, buf, sem):
            pltpu.make_async_copy(
                h_hbm.at[src_blk.at[pl.ds(c * K, K)]], buf, sem).wait()
            pltpu.sync_copy(buf, acc.at[cdst.at[c]], add=True)
            pltpu.sync_copy(ones_v, dacc.at[cdst.at[c]], add=True)

            @pl.when(c + 2 < CPB)
            def _next():
                pltpu.async_copy(
                    h_hbm.at[src_blk.at[pl.ds((c + 2) * K, K)]], buf, sem)

        def _pair(p, _):
            _step(2 * p, rows0, gsem0)
            _step(2 * p + 1, rows1, gsem1)
            return 0
        lax.fori_loop(0, CPB // 2, _pair, 0)
        _step(CPB - 1, rows0, gsem0)
        return 0
    lax.fori_loop(0, NBLK, _block, 0)
    plsc.subcore_barrier()

    wlo = sid * WPT
    pltpu.sync_copy(acc.at[pl.ds(wlo, WPT)],
                    sum_hbm.at[pl.ds(cid * N + wlo, WPT)])

    @pl.when(sid == NS - 1)
    def _tail():
        t0 = NS * WPT
        pltpu.sync_copy(acc.at[pl.ds(t0, N - t0)],
                        sum_hbm.at[pl.ds(cid * N + t0, N - t0)])

    @pl.when(sid == 0)
    def _deg_out():
        pltpu.sync_copy(dacc, deg_hbm.at[pl.ds(cid * NR, NR)])


_sc_call = pl.kernel(
    _sc_body,
    out_type=[
        jax.ShapeDtypeStruct((NC * N, Q), jnp.float32),
        jax.ShapeDtypeStruct((NC * NR,), jnp.float32),
    ],
    mesh=plsc.VectorSubcoreMesh(
        core_axis_name="c", subcore_axis_name="s", num_cores=NC,
        num_subcores=NS),
    scratch_types=[
        pltpu.VMEM((BLK,), jnp.int32),
        pltpu.VMEM((BLK,), jnp.int32),
        pltpu.VMEM((CPB, K), jnp.int32),
        pltpu.VMEM((K,), jnp.float32),
        pltpu.VMEM((K, Q), jnp.float32),
        pltpu.VMEM((K, Q), jnp.float32),
        pltpu.SemaphoreType.DMA,
        pltpu.SemaphoreType.DMA,
        pltpu.VMEM_SHARED((NR, Q), jnp.float32),
        pltpu.VMEM_SHARED((NR,), jnp.float32),
    ],
)


BR = 1000


def _tc_body(u_ref, n0_ref, n1_ref, d_ref,
             wa1, wa2, wfc1, wfc2, wfh1, wfh2, wgc1, wgc2, wz1, wz2,
             ba_r, bfc_r, bfh_r, bgc_r, bz_r, out_ref):
    c = u_ref[:, :P]
    h = u_ref[:, P:]
    nsum = n0_ref[...] + n1_ref[...]
    deg = d_ref[...]
    mean = jnp.where(deg > 0, nsum / jnp.maximum(deg, 1.0), 0.0)

    def dot(a, b):
        return lax.dot_general(a, b, (((1,), (0,)), ((), ())),
                               preferred_element_type=jnp.float32)

    def sp(x):
        return jnp.maximum(x, 0.0) + jnp.log1p(jnp.exp(-jnp.abs(x)))

    h_ = sp(dot(h, wa1[...]) + dot(mean, wa2[...]) + ba_r[...])
    fc = sp(dot(c, wfc1[...]) + dot(h_, wfc2[...]) + bfc_r[...])
    gc = sp(dot(c, wgc1[...]) + dot(h_, wgc2[...]) + bgc_r[...])
    zz = jnp.tanh(dot(c, wz1[...]) + dot(h_, wz2[...]) + bz_r[...])
    fh = sp(dot(c, wfh1[...]) + dot(h_, wfh2[...]) + bfh_r[...])
    dc = gc * zz - fc * c
    dh = -fh * h
    dc = dc - (jnp.sum(dc * c, axis=1, keepdims=True)
               / jnp.sum(c * c, axis=1, keepdims=True)) * c
    out_ref[:, :P] = dc
    out_ref[:, P:] = dh


def _full(shape):
    return pl.BlockSpec(shape, lambda i: (0,) * len(shape))


_tc_call = pl.pallas_call(
    _tc_body,
    grid=(N // BR,),
    in_specs=[
        pl.BlockSpec((BR, P + Q), lambda i: (i, 0)),
        pl.BlockSpec((BR, Q), lambda i: (i, 0)),
        pl.BlockSpec((BR, Q), lambda i: (i + N // BR, 0)),
        pl.BlockSpec((BR, 1), lambda i: (i, 0)),
    ] + [_full((P, P))] * 10 + [_full((1, P))] * 5,
    out_specs=pl.BlockSpec((BR, P + Q), lambda i: (i, 0)),
    out_shape=jax.ShapeDtypeStruct((N, P + Q), jnp.float32),
)


@jax.jit
def kernel(t, u, edge_index, Wa, ba, Wfc, bfc, Wfh, bfh, Wgc, bgc, Wz, bz):
    del t
    h_tab = u.reshape(2 * N, Q)
    src_flat = edge_index[0] * 2 + 1
    dst_flat = edge_index[1]
    z = jnp.zeros((ZR, Q), jnp.float32)
    zd = jnp.zeros((NR,), jnp.float32)
    o = jnp.ones((K,), jnp.float32)
    nsum2, deg2 = _sc_call(h_tab, src_flat, dst_flat, z, zd, o)
    deg_col = (deg2[:NR] + deg2[NR:])[:N].reshape(N, 1)
    out = _tc_call(
        u, nsum2, nsum2, deg_col,
        Wa[:Q], Wa[Q:], Wfc[:P], Wfc[P:], Wfh[:P], Wfh[P:],
        Wgc[:P], Wgc[P:], Wz[:P], Wz[P:],
        ba.reshape(1, Q), bfc.reshape(1, P), bfh.reshape(1, Q),
        bgc.reshape(1, P), bz.reshape(1, P),
    )
    return out
